# Initial kernel scaffold; baseline (speedup 1.0000x reference)
import jax
import jax.numpy as jnp
from jax.experimental import pallas as pl


def kernel(x, edge_index, edge_label_index, W_enc, b_enc, W1, b1, W2, b2, Wp1, bp1, Wp2, bp2):
    raise NotImplementedError("write your pallas kernel here")


# speedup vs baseline: 11.6695x; 11.6695x over previous
import functools

import jax
import jax.numpy as jnp
from jax import lax
from jax.experimental import pallas as pl
from jax.experimental.pallas import tpu as pltpu
from jax.experimental.pallas import tpu_sc as plsc

N = 50000
E = 800000
EL = 200000
NC, NS = 2, 16
NW = NC * NS
CHUNK = 128
KE = 200
EP = NW * KE * CHUNK
KL = 56
ELP = NW * KL * CHUNK
ACC_ROWS = 51200
APT = ACC_ROWS // NS
ZC = 128
IB = 8


def _sc_mesh():
    return plsc.VectorSubcoreMesh(
        core_axis_name="c", subcore_axis_name="s",
        num_cores=NC, num_subcores=NS)


_SC_PARAMS = pltpu.CompilerParams(use_tc_tiling_on_sc=False)


def _deg_kernel(dst2d, ones, zrows):
    @functools.partial(
        pl.kernel,
        out_type=(jax.ShapeDtypeStruct((ACC_ROWS,), jnp.float32),
                  jax.ShapeDtypeStruct((ACC_ROWS,), jnp.float32)),
        mesh=_sc_mesh(),
        compiler_params=_SC_PARAMS,
        scratch_types=[
            pltpu.VMEM((KE, CHUNK), jnp.int32),
            pltpu.VMEM((CHUNK,), jnp.float32),
            pltpu.VMEM((APT,), jnp.float32),
            pltpu.VMEM_SHARED((ACC_ROWS,), jnp.float32),
        ],
    )
    def deg(dst_hbm, ones_hbm, z_hbm, outA, outB, idx_v, ones_v, zbuf, acc):
        c = lax.axis_index("c")
        s = lax.axis_index("s")
        wid = c * NS + s
        sl = pl.ds(s * APT, APT)
        pltpu.sync_copy(z_hbm, zbuf)
        pltpu.sync_copy(zbuf, acc.at[sl])
        pltpu.sync_copy(ones_hbm, ones_v)
        pltpu.sync_copy(dst_hbm.at[pl.ds(wid * KE, KE)], idx_v)
        plsc.subcore_barrier()

        def body(j, carry):
            pltpu.sync_copy(ones_v, acc.at[idx_v.at[j]], add=True)
            return carry
        lax.fori_loop(0, KE, body, 0)
        plsc.subcore_barrier()
        pltpu.sync_copy(acc.at[sl], zbuf)

        @pl.when(c == 0)
        def _():
            pltpu.sync_copy(zbuf, outA.at[sl])

        @pl.when(c == 1)
        def _():
            pltpu.sync_copy(zbuf, outB.at[sl])

    return deg(dst2d, ones, zrows)


def _conv_agg(tA, tB, src2d, dst2d, zrows, F):
    @functools.partial(
        pl.kernel,
        out_type=(jax.ShapeDtypeStruct((ACC_ROWS, F), jnp.float32),
                  jax.ShapeDtypeStruct((ACC_ROWS, F), jnp.float32)),
        mesh=_sc_mesh(),
        compiler_params=_SC_PARAMS,
        scratch_types=[
            pltpu.VMEM((IB, CHUNK), jnp.int32),
            pltpu.VMEM((IB, CHUNK), jnp.int32),
            pltpu.VMEM((CHUNK, F), jnp.float32),
            pltpu.VMEM((ZC, F), jnp.float32),
            pltpu.VMEM_SHARED((ACC_ROWS, F), jnp.float32),
        ],
    )
    def agg(tA_hbm, tB_hbm, src_hbm, dst_hbm, z_hbm, outA, outB,
            src_v, dst_v, rows_v, zstage, acc):
        c = lax.axis_index("c")
        s = lax.axis_index("s")
        wid = c * NS + s
        pltpu.sync_copy(z_hbm, zstage)

        def zbody(k, carry):
            pltpu.sync_copy(zstage, acc.at[pl.ds(s * APT + k * ZC, ZC)])
            return carry
        lax.fori_loop(0, APT // ZC, zbody, 0)
        plsc.subcore_barrier()

        def outer(g, carry):
            gb = wid * KE + g * IB
            pltpu.sync_copy(src_hbm.at[pl.ds(gb, IB)], src_v)
            pltpu.sync_copy(dst_hbm.at[pl.ds(gb, IB)], dst_v)

            def body(j, carry2):
                @pl.when(c == 0)
                def _():
                    pltpu.sync_copy(tA_hbm.at[src_v.at[j]], rows_v)

                @pl.when(c == 1)
                def _():
                    pltpu.sync_copy(tB_hbm.at[src_v.at[j]], rows_v)

                pltpu.sync_copy(rows_v, acc.at[dst_v.at[j]], add=True)
                return carry2
            lax.fori_loop(0, IB, body, 0)
            return carry
        lax.fori_loop(0, KE // IB, outer, 0)
        plsc.subcore_barrier()

        def obody(k, carry):
            sk = pl.ds(s * APT + k * ZC, ZC)
            pltpu.sync_copy(acc.at[sk], zstage)

            @pl.when(c == 0)
            def _():
                pltpu.sync_copy(zstage, outA.at[sk])

            @pl.when(c == 1)
            def _():
                pltpu.sync_copy(zstage, outB.at[sk])
            return carry
        lax.fori_loop(0, APT // ZC, obody, 0)

    return agg(tA, tB, src2d, dst2d, zrows)


def _label_gather(h2, lsrc2d, ldst2d):
    @functools.partial(
        pl.kernel,
        out_type=jax.ShapeDtypeStruct((ELP, 32), jnp.float32),
        mesh=_sc_mesh(),
        compiler_params=_SC_PARAMS,
        scratch_types=[
            pltpu.VMEM((KL, CHUNK), jnp.int32),
            pltpu.VMEM((KL, CHUNK), jnp.int32),
            pltpu.VMEM((CHUNK, 32), jnp.float32),
        ],
    )
    def lab(h2_hbm, src_hbm, dst_hbm, ef_hbm, src_v, dst_v, buf):
        c = lax.axis_index("c")
        s = lax.axis_index("s")
        wid = c * NS + s
        pltpu.sync_copy(src_hbm.at[pl.ds(wid * KL, KL)], src_v)
        pltpu.sync_copy(dst_hbm.at[pl.ds(wid * KL, KL)], dst_v)

        def body(j, carry):
            pltpu.sync_copy(h2_hbm.at[src_v.at[j]], buf)
            pltpu.sync_copy(h2_hbm.at[dst_v.at[j]], buf, add=True)
            pltpu.sync_copy(
                buf, ef_hbm.at[pl.ds(wid * KL * CHUNK + j * CHUNK, CHUNK)])
            return carry
        lax.fori_loop(0, KL, body, 0)

    return lab(h2, lsrc2d, ldst2d)


_R = 1000


def _tc_encoder(x, W_enc, b_enc, W1):
    def body(x_ref, we_ref, be_ref, w1_ref, o_ref):
        h = jnp.dot(x_ref[...], we_ref[...],
                    preferred_element_type=jnp.float32) + be_ref[...]
        h = jnp.maximum(h, 0.0)
        o_ref[...] = jnp.dot(h, w1_ref[...],
                             preferred_element_type=jnp.float32)
    return pl.pallas_call(
        body,
        grid=(N // _R,),
        in_specs=[
            pl.BlockSpec((_R, 128), lambda i: (i, 0)),
            pl.BlockSpec((128, 64), lambda i: (0, 0)),
            pl.BlockSpec((1, 64), lambda i: (0, 0)),
            pl.BlockSpec((64, 64), lambda i: (0, 0)),
        ],
        out_specs=pl.BlockSpec((_R, 64), lambda i: (i, 0)),
        out_shape=jax.ShapeDtypeStruct((N, 64), jnp.float32),
    )(x, W_enc, b_enc.reshape(1, 64), W1)


def _tc_scale1(degp_t, xw1):
    def body(dp_ref, xw_ref, dinv_ref, ta_ref, tb_ref):
        deg = dp_ref[...][:, 0:1] + dp_ref[...][:, 1:2] + 1.0
        dinv = lax.rsqrt(deg)
        dinv_ref[...] = dinv
        t = xw_ref[...] * dinv
        ta_ref[...] = t[:, :32]
        tb_ref[...] = t[:, 32:]
    return pl.pallas_call(
        body,
        grid=(N // _R,),
        in_specs=[
            pl.BlockSpec((_R, 2), lambda i: (i, 0)),
            pl.BlockSpec((_R, 64), lambda i: (i, 0)),
        ],
        out_specs=[
            pl.BlockSpec((_R, 1), lambda i: (i, 0)),
            pl.BlockSpec((_R, 32), lambda i: (i, 0)),
            pl.BlockSpec((_R, 32), lambda i: (i, 0)),
        ],
        out_shape=[
            jax.ShapeDtypeStruct((N, 1), jnp.float32),
            jax.ShapeDtypeStruct((N, 32), jnp.float32),
            jax.ShapeDtypeStruct((N, 32), jnp.float32),
        ],
    )(degp_t, xw1)


def _tc_conv1_post(aggA, aggB, xw1, dinv, b1, W2):
    def body(aa_ref, ab_ref, xw_ref, dv_ref, b1_ref, w2_ref,
             xw2_ref, ta_ref, tb_ref):
        dv = dv_ref[...]
        agg = jnp.concatenate([aa_ref[...], ab_ref[...]], axis=1)
        h1 = dv * agg + (dv * dv) * xw_ref[...] + b1_ref[...]
        h1 = jnp.maximum(h1, 0.0)
        xw2 = jnp.dot(h1, w2_ref[...], preferred_element_type=jnp.float32)
        xw2_ref[...] = xw2
        t2 = xw2 * dv
        ta_ref[...] = t2[:, :16]
        tb_ref[...] = t2[:, 16:]
    return pl.pallas_call(
        body,
        grid=(N // _R,),
        in_specs=[
            pl.BlockSpec((_R, 32), lambda i: (i, 0)),
            pl.BlockSpec((_R, 32), lambda i: (i, 0)),
            pl.BlockSpec((_R, 64), lambda i: (i, 0)),
            pl.BlockSpec((_R, 1), lambda i: (i, 0)),
            pl.BlockSpec((1, 64), lambda i: (0, 0)),
            pl.BlockSpec((64, 32), lambda i: (0, 0)),
        ],
        out_specs=[
            pl.BlockSpec((_R, 32), lambda i: (i, 0)),
            pl.BlockSpec((_R, 16), lambda i: (i, 0)),
            pl.BlockSpec((_R, 16), lambda i: (i, 0)),
        ],
        out_shape=[
            jax.ShapeDtypeStruct((N, 32), jnp.float32),
            jax.ShapeDtypeStruct((N, 16), jnp.float32),
            jax.ShapeDtypeStruct((N, 16), jnp.float32),
        ],
    )(aggA, aggB, xw1, dinv, b1.reshape(1, 64), W2)


def _tc_conv2_post(aggA, aggB, xw2, dinv, b2):
    def body(aa_ref, ab_ref, xw_ref, dv_ref, b2_ref, o_ref):
        dv = dv_ref[...]
        agg = jnp.concatenate([aa_ref[...], ab_ref[...]], axis=1)
        o_ref[...] = dv * agg + (dv * dv) * xw_ref[...] + b2_ref[...]
    return pl.pallas_call(
        body,
        grid=(N // _R,),
        in_specs=[
            pl.BlockSpec((_R, 16), lambda i: (i, 0)),
            pl.BlockSpec((_R, 16), lambda i: (i, 0)),
            pl.BlockSpec((_R, 32), lambda i: (i, 0)),
            pl.BlockSpec((_R, 1), lambda i: (i, 0)),
            pl.BlockSpec((1, 32), lambda i: (0, 0)),
        ],
        out_specs=pl.BlockSpec((_R, 32), lambda i: (i, 0)),
        out_shape=jax.ShapeDtypeStruct((N, 32), jnp.float32),
    )(aggA, aggB, xw2, dinv, b2.reshape(1, 32))


def _tc_mlp(ef, Wp1, bp1, Wp2, bp2):
    R2 = 2048
    def body(ef_ref, w1_ref, b1_ref, w2_ref, b2_ref, o_ref):
        e = jnp.dot(ef_ref[...], w1_ref[...],
                    preferred_element_type=jnp.float32) + b1_ref[...]
        e = jnp.maximum(e, 0.0)
        z = jnp.dot(e, w2_ref[...],
                    preferred_element_type=jnp.float32) + b2_ref[...]
        o_ref[...] = 1.0 / (1.0 + jnp.exp(-z))
    return pl.pallas_call(
        body,
        grid=(ELP // R2,),
        in_specs=[
            pl.BlockSpec((R2, 32), lambda i: (i, 0)),
            pl.BlockSpec((32, 16), lambda i: (0, 0)),
            pl.BlockSpec((1, 16), lambda i: (0, 0)),
            pl.BlockSpec((16, 1), lambda i: (0, 0)),
            pl.BlockSpec((1, 1), lambda i: (0, 0)),
        ],
        out_specs=pl.BlockSpec((R2, 1), lambda i: (i, 0)),
        out_shape=jax.ShapeDtypeStruct((ELP, 1), jnp.float32),
    )(ef, Wp1, bp1.reshape(1, 16), Wp2, bp2.reshape(1, 1))


def kernel(x, edge_index, edge_label_index,
           W_enc, b_enc, W1, b1, W2, b2, Wp1, bp1, Wp2, bp2):
    f32 = jnp.float32
    i32 = jnp.int32

    src = edge_index[0]
    dst = edge_index[1]
    src_p = jnp.concatenate(
        [src, jnp.zeros((EP - E,), i32)]).reshape(EP // CHUNK, CHUNK)
    dst_p = jnp.concatenate(
        [dst, jnp.full((EP - E,), N, i32)]).reshape(EP // CHUNK, CHUNK)
    lsrc_p = jnp.concatenate(
        [edge_label_index[0], jnp.zeros((ELP - EL,), i32)]
    ).reshape(ELP // CHUNK, CHUNK)
    ldst_p = jnp.concatenate(
        [edge_label_index[1], jnp.zeros((ELP - EL,), i32)]
    ).reshape(ELP // CHUNK, CHUNK)

    z1 = jnp.zeros((APT,), f32)
    z32 = jnp.zeros((ZC, 32), f32)
    z16 = jnp.zeros((ZC, 16), f32)
    ones = jnp.ones((CHUNK,), f32)

    xw1 = _tc_encoder(x, W_enc, b_enc, W1)
    degA, degB = _deg_kernel(dst_p, ones, z1)
    degp_t = jnp.stack([degA[:N], degB[:N]], axis=1)
    dinv, tA, tB = _tc_scale1(degp_t, xw1)
    aggA, aggB = _conv_agg(tA, tB, src_p, dst_p, z32, 32)
    xw2, t2A, t2B = _tc_conv1_post(aggA[:N], aggB[:N], xw1, dinv, b1, W2)
    agg2A, agg2B = _conv_agg(t2A, t2B, src_p, dst_p, z16, 16)
    h2 = _tc_conv2_post(agg2A[:N], agg2B[:N], xw2, dinv, b2)
    ef = _label_gather(h2, lsrc_p, ldst_p)
    out = _tc_mlp(ef, Wp1, bp1, Wp2, bp2)
    return out[:EL, 0]


# speedup vs baseline: 13.5293x; 1.1594x over previous
import functools

import jax
import jax.numpy as jnp
from jax import lax
from jax.experimental import pallas as pl
from jax.experimental.pallas import tpu as pltpu
from jax.experimental.pallas import tpu_sc as plsc

N = 50000
E = 800000
EL = 200000
NC, NS = 2, 16
NW = NC * NS
CHUNK = 256
KE = 100
EP = NW * KE * CHUNK
IB = 10
KL = 28
ELP = NW * KL * CHUNK
LB = 4
ACC_ROWS = 50176
APT = ACC_ROWS // NS
ZC = 112


def _sc_mesh():
    return plsc.VectorSubcoreMesh(
        core_axis_name="c", subcore_axis_name="s",
        num_cores=NC, num_subcores=NS)


_SC_PARAMS = pltpu.CompilerParams(use_tc_tiling_on_sc=False)


def _deg_kernel(dst2d, ones, zrows):
    @functools.partial(
        pl.kernel,
        out_type=jax.ShapeDtypeStruct((NC, ACC_ROWS), jnp.float32),
        mesh=_sc_mesh(),
        compiler_params=_SC_PARAMS,
        scratch_types=[
            pltpu.VMEM((KE, CHUNK), jnp.int32),
            pltpu.VMEM((CHUNK,), jnp.float32),
            pltpu.VMEM((APT,), jnp.float32),
            pltpu.VMEM_SHARED((ACC_ROWS,), jnp.float32),
        ],
    )
    def deg(dst_hbm, ones_hbm, z_hbm, out_hbm, idx_v, ones_v, zbuf, acc):
        c = lax.axis_index("c")
        s = lax.axis_index("s")
        wid = c * NS + s
        sl = pl.ds(s * APT, APT)
        pltpu.sync_copy(z_hbm, zbuf)
        pltpu.sync_copy(zbuf, acc.at[sl])
        pltpu.sync_copy(ones_hbm, ones_v)
        pltpu.sync_copy(dst_hbm.at[pl.ds(wid * KE, KE)], idx_v)
        plsc.subcore_barrier()

        def body(j, carry):
            pltpu.sync_copy(ones_v, acc.at[idx_v.at[j]], add=True)
            return carry
        lax.fori_loop(0, KE, body, 0)
        plsc.subcore_barrier()
        pltpu.sync_copy(acc.at[sl], zbuf)
        pltpu.sync_copy(zbuf, out_hbm.at[c, sl])

    return deg(dst2d, ones, zrows)


def _conv_agg(t3, src2d, dst2d, zrows, F):
    @functools.partial(
        pl.kernel,
        out_type=jax.ShapeDtypeStruct((NC, ACC_ROWS, F), jnp.float32),
        mesh=_sc_mesh(),
        compiler_params=_SC_PARAMS,
        scratch_types=[
            pltpu.VMEM((IB, CHUNK), jnp.int32),
            pltpu.VMEM((IB, CHUNK), jnp.int32),
            pltpu.VMEM((CHUNK, F), jnp.float32),
            pltpu.VMEM((CHUNK, F), jnp.float32),
            pltpu.VMEM((ZC, F), jnp.float32),
            pltpu.SemaphoreType.DMA,
            pltpu.SemaphoreType.DMA,
            pltpu.SemaphoreType.DMA,
            pltpu.SemaphoreType.DMA,
            pltpu.VMEM_SHARED((ACC_ROWS, F), jnp.float32),
        ],
    )
    def agg(t3_hbm, src_hbm, dst_hbm, z_hbm, out_hbm,
            src_v, dst_v, rows0, rows1, zstage, sg0, sg1, ss0, ss1, acc):
        c = lax.axis_index("c")
        s = lax.axis_index("s")
        wid = c * NS + s
        tbl = t3_hbm.at[c]
        rows = (rows0, rows1)
        sg = (sg0, sg1)
        ss = (ss0, ss1)
        pltpu.sync_copy(z_hbm, zstage)

        def zbody(k, carry):
            pltpu.sync_copy(zstage, acc.at[pl.ds(s * APT + k * ZC, ZC)])
            return carry
        lax.fori_loop(0, APT // ZC, zbody, 0)
        plsc.subcore_barrier()

        def outer(g, carry):
            gb = wid * KE + g * IB
            pltpu.sync_copy(src_hbm.at[pl.ds(gb, IB)], src_v)
            pltpu.sync_copy(dst_hbm.at[pl.ds(gb, IB)], dst_v)
            dg = [None, None]
            dsc = [None, None]
            dg[0] = pltpu.async_copy(tbl.at[src_v.at[0]], rows[0], sg[0])
            for k in range(IB):
                b = k % 2
                nb = 1 - b
                dg[b].wait()
                if k + 1 < IB:
                    if dsc[nb] is not None:
                        dsc[nb].wait()
                    dg[nb] = pltpu.async_copy(
                        tbl.at[src_v.at[k + 1]], rows[nb], sg[nb])
                dsc[b] = pltpu.async_copy(
                    rows[b], acc.at[dst_v.at[k]], ss[b], add=True)
            dsc[0].wait()
            dsc[1].wait()
            return carry
        lax.fori_loop(0, KE // IB, outer, 0)
        plsc.subcore_barrier()

        def obody(k, carry):
            sk = pl.ds(s * APT + k * ZC, ZC)
            pltpu.sync_copy(acc.at[sk], zstage)
            pltpu.sync_copy(zstage, out_hbm.at[c, sk])
            return carry
        lax.fori_loop(0, APT // ZC, obody, 0)

    return agg(t3, src2d, dst2d, zrows)


def _label_gather(h2, lsrc2d, ldst2d):
    @functools.partial(
        pl.kernel,
        out_type=jax.ShapeDtypeStruct((ELP, 32), jnp.float32),
        mesh=_sc_mesh(),
        compiler_params=_SC_PARAMS,
        scratch_types=[
            pltpu.VMEM((KL, CHUNK), jnp.int32),
            pltpu.VMEM((KL, CHUNK), jnp.int32),
            pltpu.VMEM((CHUNK, 32), jnp.float32),
            pltpu.VMEM((CHUNK, 32), jnp.float32),
            pltpu.SemaphoreType.DMA,
            pltpu.SemaphoreType.DMA,
            pltpu.SemaphoreType.DMA,
            pltpu.SemaphoreType.DMA,
        ],
    )
    def lab(h2_hbm, src_hbm, dst_hbm, ef_hbm, src_v, dst_v,
            buf0, buf1, sg0, sg1, ss0, ss1):
        c = lax.axis_index("c")
        s = lax.axis_index("s")
        wid = c * NS + s
        pltpu.sync_copy(src_hbm.at[pl.ds(wid * KL, KL)], src_v)
        pltpu.sync_copy(dst_hbm.at[pl.ds(wid * KL, KL)], dst_v)
        bufs = (buf0, buf1)
        sg = (sg0, sg1)
        ss = (ss0, ss1)

        def outer(g, carry):
            jb = g * LB
            dg = [None, None]
            dst_store = [None, None]
            dg[0] = pltpu.async_copy(h2_hbm.at[src_v.at[jb]], bufs[0], sg[0])
            for k in range(LB):
                b = k % 2
                nb = 1 - b
                j = jb + k
                dg[b].wait()
                if k + 1 < LB:
                    if dst_store[nb] is not None:
                        dst_store[nb].wait()
                    dg[nb] = pltpu.async_copy(
                        h2_hbm.at[src_v.at[j + 1]], bufs[nb], sg[nb])
                pltpu.sync_copy(h2_hbm.at[dst_v.at[j]], bufs[b], add=True)
                dst_store[b] = pltpu.async_copy(
                    bufs[b],
                    ef_hbm.at[pl.ds(wid * KL * CHUNK + j * CHUNK, CHUNK)],
                    ss[b])
            dst_store[0].wait()
            dst_store[1].wait()
            return carry
        lax.fori_loop(0, KL // LB, outer, 0)

    return lab(h2, lsrc2d, ldst2d)


_R = 1024
_TG = ACC_ROWS // _R


def _tc_encoder(x, W_enc, b_enc, W1):
    def body(x_ref, we_ref, be_ref, w1_ref, o_ref):
        h = jnp.dot(x_ref[...], we_ref[...],
                    preferred_element_type=jnp.float32) + be_ref[...]
        h = jnp.maximum(h, 0.0)
        o_ref[...] = jnp.dot(h, w1_ref[...],
                             preferred_element_type=jnp.float32)
    return pl.pallas_call(
        body,
        grid=(_TG,),
        in_specs=[
            pl.BlockSpec((_R, 128), lambda i: (i, 0)),
            pl.BlockSpec((128, 64), lambda i: (0, 0)),
            pl.BlockSpec((1, 64), lambda i: (0, 0)),
            pl.BlockSpec((64, 64), lambda i: (0, 0)),
        ],
        out_specs=pl.BlockSpec((_R, 64), lambda i: (i, 0)),
        out_shape=jax.ShapeDtypeStruct((N, 64), jnp.float32),
    )(x, W_enc, b_enc.reshape(1, 64), W1)


def _tc_scale1(degp_t, xw1):
    def body(dp_ref, xw_ref, dinv_ref, ta_ref, tb_ref):
        deg = dp_ref[...][:, 0:1] + dp_ref[...][:, 1:2] + 1.0
        dinv = lax.rsqrt(deg)
        dinv_ref[...] = dinv
        t = xw_ref[...] * dinv
        ta_ref[...] = t[:, :32]
        tb_ref[...] = t[:, 32:]
    return pl.pallas_call(
        body,
        grid=(_TG,),
        in_specs=[
            pl.BlockSpec((_R, 2), lambda i: (i, 0)),
            pl.BlockSpec((_R, 64), lambda i: (i, 0)),
        ],
        out_specs=[
            pl.BlockSpec((_R, 1), lambda i: (i, 0)),
            pl.BlockSpec((_R, 32), lambda i: (i, 0)),
            pl.BlockSpec((_R, 32), lambda i: (i, 0)),
        ],
        out_shape=[
            jax.ShapeDtypeStruct((N, 1), jnp.float32),
            jax.ShapeDtypeStruct((N, 32), jnp.float32),
            jax.ShapeDtypeStruct((N, 32), jnp.float32),
        ],
    )(degp_t, xw1)


def _tc_conv1_post(agg3, xw1, dinv, b1, W2):
    def body(aa_ref, ab_ref, xw_ref, dv_ref, b1_ref, w2_ref,
             xw2_ref, ta_ref, tb_ref):
        dv = dv_ref[...]
        agg = jnp.concatenate([aa_ref[0], ab_ref[0]], axis=1)
        h1 = dv * agg + (dv * dv) * xw_ref[...] + b1_ref[...]
        h1 = jnp.maximum(h1, 0.0)
        xw2 = jnp.dot(h1, w2_ref[...], preferred_element_type=jnp.float32)
        xw2_ref[...] = xw2
        t2 = xw2 * dv
        ta_ref[...] = t2[:, :16]
        tb_ref[...] = t2[:, 16:]
    return pl.pallas_call(
        body,
        grid=(_TG,),
        in_specs=[
            pl.BlockSpec((1, _R, 32), lambda i: (0, i, 0)),
            pl.BlockSpec((1, _R, 32), lambda i: (1, i, 0)),
            pl.BlockSpec((_R, 64), lambda i: (i, 0)),
            pl.BlockSpec((_R, 1), lambda i: (i, 0)),
            pl.BlockSpec((1, 64), lambda i: (0, 0)),
            pl.BlockSpec((64, 32), lambda i: (0, 0)),
        ],
        out_specs=[
            pl.BlockSpec((_R, 32), lambda i: (i, 0)),
            pl.BlockSpec((_R, 16), lambda i: (i, 0)),
            pl.BlockSpec((_R, 16), lambda i: (i, 0)),
        ],
        out_shape=[
            jax.ShapeDtypeStruct((N, 32), jnp.float32),
            jax.ShapeDtypeStruct((N, 16), jnp.float32),
            jax.ShapeDtypeStruct((N, 16), jnp.float32),
        ],
    )(agg3, agg3, xw1, dinv, b1.reshape(1, 64), W2)


def _tc_conv2_post(agg3, xw2, dinv, b2):
    def body(aa_ref, ab_ref, xw_ref, dv_ref, b2_ref, o_ref):
        dv = dv_ref[...]
        agg = jnp.concatenate([aa_ref[0], ab_ref[0]], axis=1)
        o_ref[...] = dv * agg + (dv * dv) * xw_ref[...] + b2_ref[...]
    return pl.pallas_call(
        body,
        grid=(_TG,),
        in_specs=[
            pl.BlockSpec((1, _R, 16), lambda i: (0, i, 0)),
            pl.BlockSpec((1, _R, 16), lambda i: (1, i, 0)),
            pl.BlockSpec((_R, 32), lambda i: (i, 0)),
            pl.BlockSpec((_R, 1), lambda i: (i, 0)),
            pl.BlockSpec((1, 32), lambda i: (0, 0)),
        ],
        out_specs=pl.BlockSpec((_R, 32), lambda i: (i, 0)),
        out_shape=jax.ShapeDtypeStruct((N, 32), jnp.float32),
    )(agg3, agg3, xw2, dinv, b2.reshape(1, 32))


def _tc_mlp(ef, Wp1, bp1, Wp2, bp2):
    R2 = 2048
    def body(ef_ref, w1_ref, b1_ref, w2_ref, b2_ref, o_ref):
        e = jnp.dot(ef_ref[...], w1_ref[...],
                    preferred_element_type=jnp.float32) + b1_ref[...]
        e = jnp.maximum(e, 0.0)
        z = jnp.dot(e, w2_ref[...],
                    preferred_element_type=jnp.float32) + b2_ref[...]
        o_ref[...] = 1.0 / (1.0 + jnp.exp(-z))
    return pl.pallas_call(
        body,
        grid=(ELP // R2,),
        in_specs=[
            pl.BlockSpec((R2, 32), lambda i: (i, 0)),
            pl.BlockSpec((32, 16), lambda i: (0, 0)),
            pl.BlockSpec((1, 16), lambda i: (0, 0)),
            pl.BlockSpec((16, 1), lambda i: (0, 0)),
            pl.BlockSpec((1, 1), lambda i: (0, 0)),
        ],
        out_specs=pl.BlockSpec((R2, 1), lambda i: (i, 0)),
        out_shape=jax.ShapeDtypeStruct((ELP, 1), jnp.float32),
    )(ef, Wp1, bp1.reshape(1, 16), Wp2, bp2.reshape(1, 1))


def kernel(x, edge_index, edge_label_index,
           W_enc, b_enc, W1, b1, W2, b2, Wp1, bp1, Wp2, bp2):
    f32 = jnp.float32
    i32 = jnp.int32

    src = edge_index[0]
    dst = edge_index[1]
    src_p = jnp.concatenate(
        [src, jnp.zeros((EP - E,), i32)]).reshape(EP // CHUNK, CHUNK)
    dst_p = jnp.concatenate(
        [dst, jnp.full((EP - E,), N, i32)]).reshape(EP // CHUNK, CHUNK)
    lsrc_p = jnp.concatenate(
        [edge_label_index[0], jnp.zeros((ELP - EL,), i32)]
    ).reshape(ELP // CHUNK, CHUNK)
    ldst_p = jnp.concatenate(
        [edge_label_index[1], jnp.zeros((ELP - EL,), i32)]
    ).reshape(ELP // CHUNK, CHUNK)

    z1 = jnp.zeros((APT,), f32)
    z32 = jnp.zeros((ZC, 32), f32)
    z16 = jnp.zeros((ZC, 16), f32)
    ones = jnp.ones((CHUNK,), f32)

    xw1 = _tc_encoder(x, W_enc, b_enc, W1)
    degp = _deg_kernel(dst_p, ones, z1)
    degp_t = degp[:, :N].T
    dinv, tA, tB = _tc_scale1(degp_t, xw1)
    t3 = jnp.stack([tA, tB])
    agg3 = _conv_agg(t3, src_p, dst_p, z32, 32)
    xw2, t2A, t2B = _tc_conv1_post(agg3, xw1, dinv, b1, W2)
    t32 = jnp.stack([t2A, t2B])
    agg23 = _conv_agg(t32, src_p, dst_p, z16, 16)
    h2 = _tc_conv2_post(agg23, xw2, dinv, b2)
    ef = _label_gather(h2, lsrc_p, ldst_p)
    out = _tc_mlp(ef, Wp1, bp1, Wp2, bp2)
    return out[:EL, 0]
